# Initial kernel scaffold; baseline (speedup 1.0000x reference)
import jax
import jax.numpy as jnp
from jax.experimental import pallas as pl


def kernel(features, neighbor_idx, edge_feats, time_feats, node_idx, W1, b1, W2, b2):
    raise NotImplementedError("write your pallas kernel here")


# speedup vs baseline: 1.4474x; 1.4474x over previous
import functools

import jax
import jax.numpy as jnp
from jax import lax
from jax.experimental import pallas as pl
from jax.experimental.pallas import tpu as pltpu
from jax.experimental.pallas import tpu_sc as plsc


def _sc_gather_sum(features, nidx3, sidx3, B_pad, EMB, NW, n_chunks, CH, s_chunks, SCH, K):
    nb = B_pad // NW

    mesh = plsc.VectorSubcoreMesh(core_axis_name="c", subcore_axis_name="s")

    @functools.partial(
        pl.kernel,
        mesh=mesh,
        out_type=(
            jax.ShapeDtypeStruct((B_pad, EMB), jnp.float32),
            jax.ShapeDtypeStruct((B_pad, EMB), jnp.float32),
        ),
        scratch_types=[
            pltpu.VMEM((n_chunks, CH), jnp.int32),
            pltpu.VMEM((s_chunks, SCH), jnp.int32),
            pltpu.VMEM((CH, EMB), jnp.float32),
            pltpu.VMEM((CH, EMB), jnp.float32),
            pltpu.VMEM((nb, EMB), jnp.float32),
            pltpu.VMEM((nb, EMB), jnp.float32),
            pltpu.SemaphoreType.DMA,
            pltpu.SemaphoreType.DMA,
            pltpu.SemaphoreType.DMA,
        ],
    )
    def sc_kernel(feat_hbm, nidx_hbm, sidx_hbm, nsum_hbm, self_hbm,
                  idx_v, sidx_v, buf0, buf1, outacc, selfb, sem0, sem1, sem2):
        wid = lax.axis_index("s") * 2 + lax.axis_index("c")
        base = wid * nb

        pltpu.sync_copy(nidx_hbm.at[wid], idx_v)
        pltpu.sync_copy(sidx_hbm.at[wid], sidx_v)

        for j in range(s_chunks):
            pltpu.async_copy(feat_hbm.at[sidx_v.at[j]],
                             selfb.at[pl.ds(j * SCH, SCH)], sem2)

        def g_start(ci, buf, sem):
            pltpu.make_async_copy(feat_hbm.at[idx_v.at[ci]], buf, sem).start()

        def g_wait(ci, buf, sem):
            pltpu.make_async_copy(feat_hbm.at[idx_v.at[ci]], buf, sem).wait()

        n_vec = EMB // 16
        tgt_per_chunk = CH // K

        def accum(buf, ci):
            for bloc in range(tgt_per_chunk):
                def kbody(k4, accs, _bloc=bloc):
                    r0 = _bloc * K + k4 * 4
                    new = accs
                    for kk in range(4):
                        r = r0 + kk
                        new = tuple(new[j] + buf[r, pl.ds(j * 16, 16)]
                                    for j in range(n_vec))
                    return new
                init = tuple(jnp.zeros((16,), jnp.float32) for _ in range(n_vec))
                accs = lax.fori_loop(0, K // 4, kbody, init)
                row = ci * tgt_per_chunk + bloc
                for j in range(n_vec):
                    outacc[row, pl.ds(j * 16, 16)] = accs[j]

        g_start(0, buf0, sem0)
        g_start(1, buf1, sem1)

        def body(i, carry):
            c0 = 2 * i
            c1 = 2 * i + 1
            g_wait(c0, buf0, sem0)
            accum(buf0, c0)

            @pl.when(c0 + 2 < n_chunks)
            def _():
                g_start(c0 + 2, buf0, sem0)

            g_wait(c1, buf1, sem1)
            accum(buf1, c1)

            @pl.when(c1 + 2 < n_chunks)
            def _():
                g_start(c1 + 2, buf1, sem1)

            return carry

        lax.fori_loop(0, n_chunks // 2, body, 0)
        pltpu.sync_copy(outacc, nsum_hbm.at[pl.ds(base, nb)])

        for j in range(s_chunks):
            pltpu.make_async_copy(feat_hbm.at[sidx_v.at[j]],
                                  selfb.at[pl.ds(j * SCH, SCH)], sem2).wait()
        pltpu.sync_copy(selfb, self_hbm.at[pl.ds(base, nb)])

    return sc_kernel(features, nidx3, sidx3)


def _tc_body(e_ref, t_ref, ns_ref, sf_ref, w1a, w1b, w1c, b1r, w2a, w2b, b2r, o_ref):
    es = jnp.sum(e_ref[...], axis=1)
    ts = jnp.sum(t_ref[...], axis=1)
    pre = (jnp.dot(ns_ref[...], w1a[...], preferred_element_type=jnp.float32)
           + jnp.dot(es, w1b[...], preferred_element_type=jnp.float32)
           + jnp.dot(ts, w1c[...], preferred_element_type=jnp.float32)
           + b1r[...])
    agg = jnp.maximum(pre, 0.0)
    o_ref[...] = (jnp.dot(sf_ref[...], w2a[...], preferred_element_type=jnp.float32)
                  + jnp.dot(agg, w2b[...], preferred_element_type=jnp.float32)
                  + b2r[...])


def kernel(features, neighbor_idx, edge_feats, time_feats, node_idx, W1, b1, W2, b2):
    N, EMB = features.shape
    B, K = neighbor_idx.shape
    EDGE = edge_feats.shape[2]
    TIME = time_feats.shape[2]

    NW = 32
    CH = 128
    SCH = 64
    B_pad = 10240
    nb = B_pad // NW
    n_chunks = nb * K // CH
    s_chunks = nb // SCH

    nidx = jnp.pad(neighbor_idx.astype(jnp.int32), ((0, B_pad - B), (0, 0)))
    nidx3 = nidx.reshape(NW, n_chunks, CH)
    sidx = jnp.pad(node_idx.astype(jnp.int32), (0, B_pad - B))
    sidx3 = sidx.reshape(NW, s_chunks, SCH)

    nsum, self_feat = _sc_gather_sum(
        features, nidx3, sidx3, B_pad, EMB, NW, n_chunks, CH, s_chunks, SCH, K)

    W1T = W1.T
    w1a = W1T[:EMB]
    w1b = W1T[EMB:EMB + EDGE]
    w1c = W1T[EMB + EDGE:]
    W2T = W2.T
    w2a = W2T[:EMB]
    w2b = W2T[EMB:]
    b1r = b1.reshape(1, EMB)
    b2r = b2.reshape(1, EMB)

    BLK = 200
    grid = (B // BLK,)
    out = pl.pallas_call(
        _tc_body,
        grid=grid,
        in_specs=[
            pl.BlockSpec((BLK, K, EDGE), lambda i: (i, 0, 0)),
            pl.BlockSpec((BLK, K, TIME), lambda i: (i, 0, 0)),
            pl.BlockSpec((BLK, EMB), lambda i: (i, 0)),
            pl.BlockSpec((BLK, EMB), lambda i: (i, 0)),
            pl.BlockSpec((EMB, EMB), lambda i: (0, 0)),
            pl.BlockSpec((EDGE, EMB), lambda i: (0, 0)),
            pl.BlockSpec((TIME, EMB), lambda i: (0, 0)),
            pl.BlockSpec((1, EMB), lambda i: (0, 0)),
            pl.BlockSpec((EMB, EMB), lambda i: (0, 0)),
            pl.BlockSpec((EMB, EMB), lambda i: (0, 0)),
            pl.BlockSpec((1, EMB), lambda i: (0, 0)),
        ],
        out_specs=pl.BlockSpec((BLK, EMB), lambda i: (i, 0)),
        out_shape=jax.ShapeDtypeStruct((B, EMB), jnp.float32),
    )(edge_feats, time_feats, nsum, self_feat,
      w1a, w1b, w1c, b1r, w2a, w2b, b2r)
    return out


# speedup vs baseline: 2.0722x; 1.4316x over previous
import functools

import jax
import jax.numpy as jnp
from jax import lax
from jax.experimental import pallas as pl
from jax.experimental.pallas import tpu as pltpu
from jax.experimental.pallas import tpu_sc as plsc


def _sc_gather_sum(features, nidx3, sidx3, B_pad, EMB, NW, n_chunks, CH, s_chunks, SCH, K):
    nb = B_pad // NW

    mesh = plsc.VectorSubcoreMesh(core_axis_name="c", subcore_axis_name="s")

    @functools.partial(
        pl.kernel,
        mesh=mesh,
        out_type=(
            jax.ShapeDtypeStruct((B_pad, EMB), jnp.float32),
            jax.ShapeDtypeStruct((B_pad, EMB), jnp.float32),
        ),
        scratch_types=[
            pltpu.VMEM((n_chunks, CH), jnp.int32),
            pltpu.VMEM((s_chunks, SCH), jnp.int32),
            pltpu.VMEM((CH, EMB), jnp.float32),
            pltpu.VMEM((CH, EMB), jnp.float32),
            pltpu.VMEM((nb, EMB), jnp.float32),
            pltpu.VMEM((nb, EMB), jnp.float32),
            pltpu.SemaphoreType.DMA,
            pltpu.SemaphoreType.DMA,
            pltpu.SemaphoreType.DMA,
        ],
    )
    def sc_kernel(feat_hbm, nidx_hbm, sidx_hbm, nsum_hbm, self_hbm,
                  idx_v, sidx_v, buf0, buf1, outacc, selfb, sem0, sem1, sem2):
        wid = lax.axis_index("s") * 2 + lax.axis_index("c")
        base = wid * nb

        pltpu.sync_copy(nidx_hbm.at[wid], idx_v)
        pltpu.sync_copy(sidx_hbm.at[wid], sidx_v)

        for j in range(s_chunks):
            pltpu.async_copy(feat_hbm.at[sidx_v.at[j]],
                             selfb.at[pl.ds(j * SCH, SCH)], sem2)

        def g_start(ci, buf, sem):
            pltpu.make_async_copy(feat_hbm.at[idx_v.at[ci]], buf, sem).start()

        def g_wait(ci, buf, sem):
            pltpu.make_async_copy(feat_hbm.at[idx_v.at[ci]], buf, sem).wait()

        n_vec = EMB // 16
        tgt_per_chunk = CH // K

        def accum(buf, ci):
            for bloc in range(tgt_per_chunk):
                def kbody(k4, accs, _bloc=bloc):
                    r0 = _bloc * K + k4 * 4
                    new = accs
                    for kk in range(4):
                        r = r0 + kk
                        new = tuple(new[j] + buf[r, pl.ds(j * 16, 16)]
                                    for j in range(n_vec))
                    return new
                init = tuple(jnp.zeros((16,), jnp.float32) for _ in range(n_vec))
                accs = lax.fori_loop(0, K // 4, kbody, init)
                row = ci * tgt_per_chunk + bloc
                for j in range(n_vec):
                    outacc[row, pl.ds(j * 16, 16)] = accs[j]

        g_start(0, buf0, sem0)
        g_start(1, buf1, sem1)

        def body(i, carry):
            c0 = 2 * i
            c1 = 2 * i + 1
            g_wait(c0, buf0, sem0)
            accum(buf0, c0)

            @pl.when(c0 + 2 < n_chunks)
            def _():
                g_start(c0 + 2, buf0, sem0)

            g_wait(c1, buf1, sem1)
            accum(buf1, c1)

            @pl.when(c1 + 2 < n_chunks)
            def _():
                g_start(c1 + 2, buf1, sem1)

            return carry

        lax.fori_loop(0, n_chunks // 2, body, 0)
        pltpu.sync_copy(outacc, nsum_hbm.at[pl.ds(base, nb)])

        for j in range(s_chunks):
            pltpu.make_async_copy(feat_hbm.at[sidx_v.at[j]],
                                  selfb.at[pl.ds(j * SCH, SCH)], sem2).wait()
        pltpu.sync_copy(selfb, self_hbm.at[pl.ds(base, nb)])

    return sc_kernel(features, nidx3, sidx3)


def _tc_body(e_ref, t_ref, ns_ref, sf_ref, w1a, w1b, w1c, b1r, w2a, w2b, b2r, o_ref):
    es = jnp.sum(e_ref[...], axis=0)
    ts = jnp.sum(t_ref[...], axis=1)
    dn = (((0,), (0,)), ((), ()))
    pre = (jnp.dot(ns_ref[...], w1a[...], preferred_element_type=jnp.float32)
           + lax.dot_general(es, w1b[...], dn, preferred_element_type=jnp.float32)
           + lax.dot_general(ts, w1c[...], dn, preferred_element_type=jnp.float32)
           + b1r[...])
    agg = jnp.maximum(pre, 0.0)
    o_ref[...] = (jnp.dot(sf_ref[...], w2a[...], preferred_element_type=jnp.float32)
                  + jnp.dot(agg, w2b[...], preferred_element_type=jnp.float32)
                  + b2r[...])


def kernel(features, neighbor_idx, edge_feats, time_feats, node_idx, W1, b1, W2, b2):
    N, EMB = features.shape
    B, K = neighbor_idx.shape
    EDGE = edge_feats.shape[2]
    TIME = time_feats.shape[2]

    NW = 32
    CH = 128
    SCH = 64
    B_pad = 10240
    nb = B_pad // NW
    n_chunks = nb * K // CH
    s_chunks = nb // SCH

    nidx = jnp.pad(neighbor_idx.astype(jnp.int32), ((0, B_pad - B), (0, 0)))
    nidx3 = nidx.reshape(NW, n_chunks, CH)
    sidx = jnp.pad(node_idx.astype(jnp.int32), (0, B_pad - B))
    sidx3 = sidx.reshape(NW, s_chunks, SCH)

    nsum, self_feat = _sc_gather_sum(
        features, nidx3, sidx3, B_pad, EMB, NW, n_chunks, CH, s_chunks, SCH, K)

    W1T = W1.T
    w1a = W1T[:EMB]
    w1b = W1T[EMB:EMB + EDGE]
    w1c = W1T[EMB + EDGE:]
    W2T = W2.T
    w2a = W2T[:EMB]
    w2b = W2T[EMB:]
    b1r = b1.reshape(1, EMB)
    b2r = b2.reshape(1, EMB)

    et = jnp.transpose(edge_feats, (1, 2, 0))
    tt = jnp.transpose(time_feats, (2, 1, 0))

    BLK = 512
    grid = ((B + BLK - 1) // BLK,)
    out = pl.pallas_call(
        _tc_body,
        grid=grid,
        in_specs=[
            pl.BlockSpec((K, EDGE, BLK), lambda i: (0, 0, i)),
            pl.BlockSpec((TIME, K, BLK), lambda i: (0, 0, i)),
            pl.BlockSpec((BLK, EMB), lambda i: (i, 0)),
            pl.BlockSpec((BLK, EMB), lambda i: (i, 0)),
            pl.BlockSpec((EMB, EMB), lambda i: (0, 0)),
            pl.BlockSpec((EDGE, EMB), lambda i: (0, 0)),
            pl.BlockSpec((TIME, EMB), lambda i: (0, 0)),
            pl.BlockSpec((1, EMB), lambda i: (0, 0)),
            pl.BlockSpec((EMB, EMB), lambda i: (0, 0)),
            pl.BlockSpec((EMB, EMB), lambda i: (0, 0)),
            pl.BlockSpec((1, EMB), lambda i: (0, 0)),
        ],
        out_specs=pl.BlockSpec((BLK, EMB), lambda i: (i, 0)),
        out_shape=jax.ShapeDtypeStruct((B, EMB), jnp.float32),
    )(et, tt, nsum, self_feat,
      w1a, w1b, w1c, b1r, w2a, w2b, b2r)
    return out
